# baseline (speedup 1.0000x reference)
import functools

import jax
import jax.numpy as jnp
from jax import lax
from jax.experimental import pallas as pl
from jax.experimental.pallas import tpu as pltpu
from jax.experimental.pallas import tpu_sc as plsc

NC = 1
NS = 16
NW = NC * NS
B = 16384
D = 20
R = 8
CPT = B // NW
L = 16
NGRP = CPT // L

_mesh = plsc.VectorSubcoreMesh(core_axis_name="c", subcore_axis_name="s", num_cores=1)


@functools.partial(
    pl.kernel,
    mesh=_mesh,
    out_type=jax.ShapeDtypeStruct((B, D), jnp.float32),
    scratch_types=[
        pltpu.VMEM((R, D), jnp.float32),
        pltpu.VMEM((CPT,), jnp.int32),
        pltpu.VMEM((CPT, D), jnp.float32),
        pltpu.SemaphoreType.DMA,
        pltpu.SemaphoreType.DMA,
    ],
    compiler_params=pltpu.CompilerParams(
        use_tc_tiling_on_sc=False,
        needs_layout_passes=False,
        disable_bounds_checks=True,
        disable_semaphore_checks=True,
        skip_device_barrier=True,
    ),
)
def _embed_gather(labels_hbm, table_hbm, out_hbm, table_v, idx_v, out_v, sem_in, sem_out):
    wid = lax.axis_index("s") * NC + lax.axis_index("c")
    base = wid * CPT
    cp_tab = pltpu.async_copy(table_hbm, table_v, sem_in)
    cp_idx = pltpu.async_copy(labels_hbm.at[pl.ds(base, CPT)], idx_v, sem_in)
    cp_tab.wait()

    lanes = lax.iota(jnp.int32, L)

    cols = [
        plsc.load_gather(table_v, [lanes & 7, jnp.full((L,), j, jnp.int32)])
        for j in range(D)
    ]
    cp_idx.wait()

    def body(g, carry):
        lbls = idx_v[pl.ds(g * L, L)]
        rows = g * L + lanes
        for j in range(D):
            vals = cols[j].at[lbls].get(mode=lax.GatherScatterMode.PROMISE_IN_BOUNDS)
            plsc.store_scatter(out_v, [rows, jnp.full((L,), j, jnp.int32)], vals)
        return carry

    SPLIT = (3 * NGRP) // 4
    SC_ = SPLIT * L
    lax.fori_loop(0, SPLIT, body, 0)
    cp0 = pltpu.async_copy(
        out_v.at[pl.ds(0, SC_)], out_hbm.at[pl.ds(base, SC_)], sem_out
    )
    lax.fori_loop(SPLIT, NGRP, body, 0)
    cp1 = pltpu.async_copy(
        out_v.at[pl.ds(SC_, CPT - SC_)],
        out_hbm.at[pl.ds(base + SC_, CPT - SC_)],
        sem_out,
    )
    cp0.wait()
    cp1.wait()


def kernel(labels, table):
    return _embed_gather(labels.astype(jnp.int32), table)


# speedup vs baseline: 1.0066x; 1.0066x over previous
import functools

import jax
import jax.numpy as jnp
from jax import lax
from jax.experimental import pallas as pl
from jax.experimental.pallas import tpu as pltpu
from jax.experimental.pallas import tpu_sc as plsc

NC = 1
NS = 16
NW = NC * NS
B = 16384
D = 20
R = 8
CPT = B // NW
L = 16
NGRP = CPT // L

_mesh = plsc.VectorSubcoreMesh(core_axis_name="c", subcore_axis_name="s", num_cores=1)


@functools.partial(
    pl.kernel,
    mesh=_mesh,
    out_type=jax.ShapeDtypeStruct((B, D), jnp.float32),
    scratch_types=[
        pltpu.VMEM((R, D), jnp.float32),
        pltpu.VMEM((CPT,), jnp.int32),
        pltpu.VMEM((CPT, D), jnp.float32),
        pltpu.SemaphoreType.DMA,
        pltpu.SemaphoreType.DMA,
    ],
    compiler_params=pltpu.CompilerParams(
        use_tc_tiling_on_sc=False,
        needs_layout_passes=False,
        disable_bounds_checks=True,
        disable_semaphore_checks=True,
        skip_device_barrier=True,
    ),
)
def _embed_gather(labels_hbm, table_hbm, out_hbm, table_v, idx_v, out_v, sem_in, sem_out):
    wid = lax.axis_index("s") * NC + lax.axis_index("c")
    base = wid * CPT
    cp_tab = pltpu.async_copy(table_hbm, table_v, sem_in)
    cp_idx = pltpu.async_copy(labels_hbm.at[pl.ds(base, CPT)], idx_v, sem_in)
    cp_tab.wait()

    lanes = lax.iota(jnp.int32, L)
    HALF = NGRP // 2

    cols = [
        plsc.load_gather(table_v, [lanes & 7, jnp.full((L,), j, jnp.int32)])
        for j in range(D)
    ]
    cp_idx.wait()

    def body(g, carry):
        lbls = idx_v[pl.ds(g * L, L)]
        rows = g * L + lanes
        for j in range(D):
            vals = cols[j].at[lbls].get(mode=lax.GatherScatterMode.PROMISE_IN_BOUNDS)
            plsc.store_scatter(out_v, [rows, jnp.full((L,), j, jnp.int32)], vals)
        return carry

    lax.fori_loop(0, HALF, body, 0)
    cp0 = pltpu.async_copy(
        out_v.at[pl.ds(0, CPT // 2)], out_hbm.at[pl.ds(base, CPT // 2)], sem_out
    )
    lax.fori_loop(HALF, NGRP, body, 0)
    cp1 = pltpu.async_copy(
        out_v.at[pl.ds(CPT // 2, CPT // 2)],
        out_hbm.at[pl.ds(base + CPT // 2, CPT // 2)],
        sem_out,
    )
    cp0.wait()
    cp1.wait()


def kernel(labels, table):
    return _embed_gather(labels.astype(jnp.int32), table)
